# Initial kernel scaffold; baseline (speedup 1.0000x reference)
import jax
import jax.numpy as jnp
from jax.experimental import pallas as pl


def kernel(emb, edge_index):
    raise NotImplementedError("write your pallas kernel here")


# speedup vs baseline: 10.8148x; 10.8148x over previous
import functools

import jax
import jax.numpy as jnp
from jax import lax
from jax.experimental import pallas as pl
from jax.experimental.pallas import tpu as pltpu
from jax.experimental.pallas import tpu_sc as plsc

N = 50000
D = 64
H = 32
NUM_LAYERS = 3
ALPHA = 1.0 / (NUM_LAYERS + 1)

NC = 2
NS = 16

NP = 50176
NT = NP // NS
CN = 112
NQ = NT // CN

E = 800000
EPT = 50176
E_PAD = EPT * NS
BLK = 128
K = 2
CB = EPT // BLK
NJ = CB // K
RB = E_PAD // BLK


def _rsqrt16(d):
    y = jnp.full((16,), 2.0 ** -11, jnp.float32)
    for k in range(9, -1, -1):
        y = jnp.where(d < 4.0 ** (k + 1), jnp.float32(2.0 ** -(k + 1)), y)
    for _ in range(6):
        y = y * (1.5 - 0.5 * d * y * y)
    return jnp.where(d > 0.5, y, 0.0)


def _propagate_body(emb_flat, row2d, col2d, o_flat, y_hbm,
                    xb, ob, yb, dinvv, idx_r, idx_c, rows, ones_v,
                    acc_sh, deg_sh, gsem):
    c = lax.axis_index("c")
    s = lax.axis_index("s")
    z16 = jnp.zeros((16,), jnp.float32)
    one16 = jnp.ones((16,), jnp.float32)

    def _zero_yb(i, _):
        yb[i, 0:16] = z16
        yb[i, 16:32] = z16
        return 0

    def _fill_ones(k, _):
        ones_v[pl.ds(k * 16, 16)] = one16
        return 0
    lax.fori_loop(0, BLK // 16, _fill_ones, 0)

    def _zero_dinvv(k, _):
        dinvv[pl.ds(k * 16, 16)] = z16
        return 0
    lax.fori_loop(0, NT // 16, _zero_dinvv, 0)
    pltpu.sync_copy(dinvv, deg_sh.at[pl.ds(s * NT, NT)])
    plsc.subcore_barrier()

    def _deg_chunk(j, _):
        cblk = s * CB + j * K
        pltpu.sync_copy(col2d.at[pl.ds(cblk, K)], idx_c)
        for t in range(K):
            pltpu.sync_copy(ones_v, deg_sh.at[idx_c.at[t]], add=True)
        return 0
    lax.fori_loop(0, NJ, _deg_chunk, 0)
    plsc.subcore_barrier()

    pltpu.sync_copy(deg_sh.at[pl.ds(s * NT, NT)], dinvv)

    def _dinv(k, _):
        dinvv[pl.ds(k * 16, 16)] = _rsqrt16(dinvv[pl.ds(k * 16, 16)])
        return 0
    lax.fori_loop(0, NT // 16, _dinv, 0)

    def _p0(q, _):
        g = c * NP + s * NT + q * CN
        pltpu.sync_copy(emb_flat.at[pl.ds(g, CN)], xb)

        def _n(i, _):
            li = q * CN + i
            dv = plsc.load_gather(dinvv, [jnp.full((16,), li, jnp.int32)])
            yb[i, 0:16] = xb[i, 0:16] * dv
            yb[i, 16:32] = xb[i, 16:32] * dv
            return 0
        lax.fori_loop(0, CN, _n, 0)
        pltpu.sync_copy(xb, o_flat.at[pl.ds(g, CN)])
        pltpu.sync_copy(yb, y_hbm.at[pl.ds(g, CN)])
        return 0
    lax.fori_loop(0, NQ, _p0, 0)

    for l in range(NUM_LAYERS):
        last = l == NUM_LAYERS - 1

        lax.fori_loop(0, CN, _zero_yb, 0)

        def _zero_acc(q, _):
            pltpu.sync_copy(yb, acc_sh.at[pl.ds(s * NT + q * CN, CN)])
            return 0
        lax.fori_loop(0, NQ, _zero_acc, 0)
        plsc.subcore_barrier()

        def _edge_chunk(j, _):
            cblk = s * CB + j * K
            rblk = c * RB + cblk
            pltpu.sync_copy(row2d.at[pl.ds(rblk, K)], idx_r)
            pltpu.sync_copy(col2d.at[pl.ds(cblk, K)], idx_c)
            descs = [pltpu.async_copy(y_hbm.at[idx_r.at[t]], rows.at[t], gsem)
                     for t in range(K)]
            for dsc in descs:
                dsc.wait()
            for t in range(K):
                pltpu.sync_copy(rows.at[t], acc_sh.at[idx_c.at[t]], add=True)
            return 0
        lax.fori_loop(0, NJ, _edge_chunk, 0)
        plsc.subcore_barrier()

        def _npass(q, _):
            g = c * NP + s * NT + q * CN
            pltpu.sync_copy(acc_sh.at[pl.ds(s * NT + q * CN, CN)], xb)
            pltpu.sync_copy(o_flat.at[pl.ds(g, CN)], ob)

            def _n(i, _):
                li = q * CN + i
                dv = plsc.load_gather(dinvv, [jnp.full((16,), li, jnp.int32)])
                x0 = xb[i, 0:16] * dv
                x1 = xb[i, 16:32] * dv
                o0 = ob[i, 0:16] + x0
                o1 = ob[i, 16:32] + x1
                if last:
                    o0 = o0 * ALPHA
                    o1 = o1 * ALPHA
                else:
                    yb[i, 0:16] = x0 * dv
                    yb[i, 16:32] = x1 * dv
                ob[i, 0:16] = o0
                ob[i, 16:32] = o1
                return 0
            lax.fori_loop(0, CN, _n, 0)
            pltpu.sync_copy(ob, o_flat.at[pl.ds(g, CN)])
            if not last:
                pltpu.sync_copy(yb, y_hbm.at[pl.ds(g, CN)])
            return 0
        lax.fori_loop(0, NQ, _npass, 0)


_propagate = functools.partial(
    pl.kernel,
    out_type=[
        jax.ShapeDtypeStruct((2 * NP, H), jnp.float32),
        jax.ShapeDtypeStruct((2 * NP, H), jnp.float32),
    ],
    mesh=plsc.VectorSubcoreMesh(core_axis_name="c", subcore_axis_name="s"),
    compiler_params=pltpu.CompilerParams(
        needs_layout_passes=False, use_tc_tiling_on_sc=False),
    scratch_types=[
        pltpu.VMEM((CN, H), jnp.float32),
        pltpu.VMEM((CN, H), jnp.float32),
        pltpu.VMEM((CN, H), jnp.float32),
        pltpu.VMEM((NT,), jnp.float32),
        pltpu.VMEM((K, BLK), jnp.int32),
        pltpu.VMEM((K, BLK), jnp.int32),
        pltpu.VMEM((K, BLK, H), jnp.float32),
        pltpu.VMEM((BLK,), jnp.float32),
        pltpu.VMEM_SHARED((NP, H), jnp.float32),
        pltpu.VMEM_SHARED((NP,), jnp.float32),
        pltpu.SemaphoreType.DMA,
    ],
)(_propagate_body)


def kernel(emb, edge_index):
    row = edge_index[0].astype(jnp.int32)
    col = edge_index[1].astype(jnp.int32)
    pad = E_PAD - E
    rp = jnp.concatenate([row, jnp.zeros((pad,), jnp.int32)])
    cp = jnp.concatenate([col, jnp.full((pad,), N, jnp.int32)])
    row2d = jnp.concatenate([rp, rp + NP]).reshape(2 * RB, BLK)
    col2d = cp.reshape(RB, BLK)
    emb_flat = jnp.zeros((2 * NP, H), jnp.float32)
    emb_flat = emb_flat.at[0:N].set(emb[:, :H]).at[NP:NP + N].set(emb[:, H:])
    o_flat, _ = _propagate(emb_flat, row2d, col2d)
    return jnp.concatenate([o_flat[:N], o_flat[NP:NP + N]], axis=1)


# speedup vs baseline: 12.9072x; 1.1935x over previous
import functools

import jax
import jax.numpy as jnp
from jax import lax
from jax.experimental import pallas as pl
from jax.experimental.pallas import tpu as pltpu
from jax.experimental.pallas import tpu_sc as plsc

N = 50000
D = 64
H = 32
NUM_LAYERS = 3
ALPHA = 1.0 / (NUM_LAYERS + 1)

NC = 2
NS = 16

NP = 50176
NT = NP // NS
CN = 112
NQ = NT // CN

E = 800000
EPT = 50176
E_PAD = EPT * NS
BLK = 128
G = 8
CB = EPT // BLK
NJ = CB // G
RB = E_PAD // BLK


def _rsqrt16(d):
    y = jnp.full((16,), 2.0 ** -11, jnp.float32)
    for k in range(9, -1, -1):
        y = jnp.where(d < 4.0 ** (k + 1), jnp.float32(2.0 ** -(k + 1)), y)
    for _ in range(6):
        y = y * (1.5 - 0.5 * d * y * y)
    return jnp.where(d > 0.5, y, 0.0)


def _propagate_body(emb_flat, row2d, col2d, o_flat, y_hbm,
                    xb, ob, yb, dinvv, idx_r, idx_c, rows, ones_v,
                    acc_sh, deg_sh, gsem, ssem):
    c = lax.axis_index("c")
    s = lax.axis_index("s")
    z16 = jnp.zeros((16,), jnp.float32)
    one16 = jnp.ones((16,), jnp.float32)

    def _zero_yb(i, _):
        yb[i, 0:16] = z16
        yb[i, 16:32] = z16
        return 0

    def _fill_ones(k, _):
        ones_v[pl.ds(k * 16, 16)] = one16
        return 0
    lax.fori_loop(0, BLK // 16, _fill_ones, 0)

    def _zero_dinvv(k, _):
        dinvv[pl.ds(k * 16, 16)] = z16
        return 0
    lax.fori_loop(0, NT // 16, _zero_dinvv, 0)
    pltpu.sync_copy(dinvv, deg_sh.at[pl.ds(s * NT, NT)])
    plsc.subcore_barrier()

    def _deg_chunk(j, _):
        cblk = s * CB + j * G
        pltpu.sync_copy(col2d.at[pl.ds(cblk, G)], idx_c)
        for t in range(G):
            pltpu.sync_copy(ones_v, deg_sh.at[idx_c.at[t]], add=True)
        return 0
    lax.fori_loop(0, NJ, _deg_chunk, 0)
    plsc.subcore_barrier()

    pltpu.sync_copy(deg_sh.at[pl.ds(s * NT, NT)], dinvv)

    def _dinv(k, _):
        dinvv[pl.ds(k * 16, 16)] = _rsqrt16(dinvv[pl.ds(k * 16, 16)])
        return 0
    lax.fori_loop(0, NT // 16, _dinv, 0)

    def _p0(q, _):
        g = c * NP + s * NT + q * CN
        pltpu.sync_copy(emb_flat.at[pl.ds(g, CN)], xb)

        def _n(i, _):
            li = q * CN + i
            dv = plsc.load_gather(dinvv, [jnp.full((16,), li, jnp.int32)])
            yb[i, 0:16] = xb[i, 0:16] * dv
            yb[i, 16:32] = xb[i, 16:32] * dv
            return 0
        lax.fori_loop(0, CN, _n, 0)
        pltpu.sync_copy(xb, o_flat.at[pl.ds(g, CN)])
        pltpu.sync_copy(yb, y_hbm.at[pl.ds(g, CN)])
        return 0
    lax.fori_loop(0, NQ, _p0, 0)

    for l in range(NUM_LAYERS):
        last = l == NUM_LAYERS - 1

        lax.fori_loop(0, CN, _zero_yb, 0)

        def _zero_acc(q, _):
            pltpu.sync_copy(yb, acc_sh.at[pl.ds(s * NT + q * CN, CN)])
            return 0
        lax.fori_loop(0, NQ, _zero_acc, 0)
        plsc.subcore_barrier()

        def _edge_chunk(j, _):
            cblk = s * CB + j * G
            rblk = c * RB + cblk
            pltpu.sync_copy(row2d.at[pl.ds(rblk, G)], idx_r)
            pltpu.sync_copy(col2d.at[pl.ds(cblk, G)], idx_c)
            sd = [None] * G
            for t in range(G):
                b = t % 2
                if t >= 2:
                    sd[t - 2].wait()
                pltpu.async_copy(y_hbm.at[idx_r.at[t]], rows.at[b],
                                 gsem).wait()
                sd[t] = pltpu.async_copy(rows.at[b], acc_sh.at[idx_c.at[t]],
                                         ssem, add=True)
            sd[G - 2].wait()
            sd[G - 1].wait()
            return 0
        lax.fori_loop(0, NJ, _edge_chunk, 0)
        plsc.subcore_barrier()

        def _npass(q, _):
            g = c * NP + s * NT + q * CN
            pltpu.sync_copy(acc_sh.at[pl.ds(s * NT + q * CN, CN)], xb)
            pltpu.sync_copy(o_flat.at[pl.ds(g, CN)], ob)

            def _n(i, _):
                li = q * CN + i
                dv = plsc.load_gather(dinvv, [jnp.full((16,), li, jnp.int32)])
                x0 = xb[i, 0:16] * dv
                x1 = xb[i, 16:32] * dv
                o0 = ob[i, 0:16] + x0
                o1 = ob[i, 16:32] + x1
                if last:
                    o0 = o0 * ALPHA
                    o1 = o1 * ALPHA
                else:
                    yb[i, 0:16] = x0 * dv
                    yb[i, 16:32] = x1 * dv
                ob[i, 0:16] = o0
                ob[i, 16:32] = o1
                return 0
            lax.fori_loop(0, CN, _n, 0)
            pltpu.sync_copy(ob, o_flat.at[pl.ds(g, CN)])
            if not last:
                pltpu.sync_copy(yb, y_hbm.at[pl.ds(g, CN)])
            return 0
        lax.fori_loop(0, NQ, _npass, 0)


_propagate = functools.partial(
    pl.kernel,
    out_type=[
        jax.ShapeDtypeStruct((2 * NP, H), jnp.float32),
        jax.ShapeDtypeStruct((2 * NP, H), jnp.float32),
    ],
    mesh=plsc.VectorSubcoreMesh(core_axis_name="c", subcore_axis_name="s"),
    compiler_params=pltpu.CompilerParams(
        needs_layout_passes=False, use_tc_tiling_on_sc=False),
    scratch_types=[
        pltpu.VMEM((CN, H), jnp.float32),
        pltpu.VMEM((CN, H), jnp.float32),
        pltpu.VMEM((CN, H), jnp.float32),
        pltpu.VMEM((NT,), jnp.float32),
        pltpu.VMEM((G, BLK), jnp.int32),
        pltpu.VMEM((G, BLK), jnp.int32),
        pltpu.VMEM((2, BLK, H), jnp.float32),
        pltpu.VMEM((BLK,), jnp.float32),
        pltpu.VMEM_SHARED((NP, H), jnp.float32),
        pltpu.VMEM_SHARED((NP,), jnp.float32),
        pltpu.SemaphoreType.DMA,
        pltpu.SemaphoreType.DMA,
    ],
)(_propagate_body)


def kernel(emb, edge_index):
    row = edge_index[0].astype(jnp.int32)
    col = edge_index[1].astype(jnp.int32)
    pad = E_PAD - E
    rp = jnp.concatenate([row, jnp.zeros((pad,), jnp.int32)])
    cp = jnp.concatenate([col, jnp.full((pad,), N, jnp.int32)])
    row2d = jnp.concatenate([rp, rp + NP]).reshape(2 * RB, BLK)
    col2d = cp.reshape(RB, BLK)
    emb_flat = jnp.zeros((2 * NP, H), jnp.float32)
    emb_flat = emb_flat.at[0:N].set(emb[:, :H]).at[NP:NP + N].set(emb[:, H:])
    o_flat, _ = _propagate(emb_flat, row2d, col2d)
    return jnp.concatenate([o_flat[:N], o_flat[NP:NP + N]], axis=1)


# speedup vs baseline: 17.1781x; 1.3309x over previous
import functools

import jax
import jax.numpy as jnp
from jax import lax
from jax.experimental import pallas as pl
from jax.experimental.pallas import tpu as pltpu
from jax.experimental.pallas import tpu_sc as plsc

N = 50000
D = 64
H = 32
NUM_LAYERS = 3
ALPHA = 1.0 / (NUM_LAYERS + 1)

NC = 2
NS = 16

NP = 50176
NT = NP // NS
CN = 64
NQ = NT // CN
S = 4

E = 800000
EPT = 50176
E_PAD = EPT * NS
BLK = 128
G = 8
CB = EPT // BLK
NJ = CB // G
RB = E_PAD // BLK


def _rsqrt16(d):
    y = jnp.full((16,), 2.0 ** -11, jnp.float32)
    for k in range(9, -1, -1):
        y = jnp.where(d < 4.0 ** (k + 1), jnp.float32(2.0 ** -(k + 1)), y)
    for _ in range(6):
        y = y * (1.5 - 0.5 * d * y * y)
    return jnp.where(d > 0.5, y, 0.0)


def _propagate_body(emb_flat, row2d, col2d, o_flat, y_hbm,
                    xb, ob, dinvv, idx_r, idx_c, rows, ones_v,
                    acc_sh, deg_sh, gsem, ssem):
    c = lax.axis_index("c")
    s = lax.axis_index("s")
    z16 = jnp.zeros((16,), jnp.float32)
    one16 = jnp.ones((16,), jnp.float32)

    def _zero_ob(i, _):
        ob[i, 0:16] = z16
        ob[i, 16:32] = z16
        return 0

    def _fill_ones(k, _):
        ones_v[pl.ds(k * 16, 16)] = one16
        return 0
    lax.fori_loop(0, BLK // 16, _fill_ones, 0)

    def _zero_dinvv(k, _):
        dinvv[pl.ds(k * 16, 16)] = z16
        return 0
    lax.fori_loop(0, NT // 16, _zero_dinvv, 0)
    pltpu.sync_copy(dinvv, deg_sh.at[pl.ds(s * NT, NT)])
    plsc.subcore_barrier()

    def _deg_chunk(j, _):
        cblk = s * CB + j * G
        pltpu.sync_copy(col2d.at[pl.ds(cblk, G)], idx_c)
        for t in range(G):
            pltpu.sync_copy(ones_v, deg_sh.at[idx_c.at[t]], add=True)
        return 0
    lax.fori_loop(0, NJ, _deg_chunk, 0)
    plsc.subcore_barrier()

    pltpu.sync_copy(deg_sh.at[pl.ds(s * NT, NT)], dinvv)

    def _dinv(k, _):
        dinvv[pl.ds(k * 16, 16)] = _rsqrt16(dinvv[pl.ds(k * 16, 16)])
        return 0
    lax.fori_loop(0, NT // 16, _dinv, 0)

    def _p0(q, _):
        g = c * NP + s * NT + q * CN
        pltpu.sync_copy(emb_flat.at[pl.ds(g, CN)], xb)
        pltpu.sync_copy(xb, o_flat.at[pl.ds(g, CN)])

        def _n(i, _):
            li = q * CN + i
            dv = plsc.load_gather(dinvv, [jnp.full((16,), li, jnp.int32)])
            xb[i, 0:16] = xb[i, 0:16] * dv
            xb[i, 16:32] = xb[i, 16:32] * dv
            return 0
        lax.fori_loop(0, CN, _n, 0)
        pltpu.sync_copy(xb, y_hbm.at[pl.ds(g, CN)])
        return 0
    lax.fori_loop(0, NQ, _p0, 0)

    for l in range(NUM_LAYERS):
        last = l == NUM_LAYERS - 1

        lax.fori_loop(0, CN, _zero_ob, 0)

        def _zero_acc(q, _):
            pltpu.sync_copy(ob, acc_sh.at[pl.ds(s * NT + q * CN, CN)])
            return 0
        lax.fori_loop(0, NQ, _zero_acc, 0)
        plsc.subcore_barrier()

        def _edge_chunk(j, _):
            cblk = s * CB + j * G
            rblk = c * RB + cblk
            pltpu.sync_copy(row2d.at[pl.ds(rblk, G)], idx_r)
            pltpu.sync_copy(col2d.at[pl.ds(cblk, G)], idx_c)
            gd = [None] * G
            sd = [None] * G
            for t in range(G):
                if t >= S:
                    sd[t - S].wait()
                gd[t] = pltpu.async_copy(y_hbm.at[idx_r.at[t]],
                                         rows.at[t % S], gsem)
                u = t - (S - 1)
                if u >= 0:
                    gd[u].wait()
                    sd[u] = pltpu.async_copy(rows.at[u % S],
                                             acc_sh.at[idx_c.at[u]],
                                             ssem, add=True)
            for u in range(G - (S - 1), G):
                gd[u].wait()
                sd[u] = pltpu.async_copy(rows.at[u % S],
                                         acc_sh.at[idx_c.at[u]],
                                         ssem, add=True)
            for u in range(G - S, G):
                if u >= 0:
                    sd[u].wait()
            return 0
        lax.fori_loop(0, NJ, _edge_chunk, 0)
        plsc.subcore_barrier()

        def _npass(q, _):
            g = c * NP + s * NT + q * CN
            pltpu.sync_copy(acc_sh.at[pl.ds(s * NT + q * CN, CN)], xb)
            pltpu.sync_copy(o_flat.at[pl.ds(g, CN)], ob)

            def _n(i, _):
                li = q * CN + i
                dv = plsc.load_gather(dinvv, [jnp.full((16,), li, jnp.int32)])
                x0 = xb[i, 0:16] * dv
                x1 = xb[i, 16:32] * dv
                o0 = ob[i, 0:16] + x0
                o1 = ob[i, 16:32] + x1
                if last:
                    o0 = o0 * ALPHA
                    o1 = o1 * ALPHA
                else:
                    xb[i, 0:16] = x0 * dv
                    xb[i, 16:32] = x1 * dv
                ob[i, 0:16] = o0
                ob[i, 16:32] = o1
                return 0
            lax.fori_loop(0, CN, _n, 0)
            pltpu.sync_copy(ob, o_flat.at[pl.ds(g, CN)])
            if not last:
                pltpu.sync_copy(xb, y_hbm.at[pl.ds(g, CN)])
            return 0
        lax.fori_loop(0, NQ, _npass, 0)


_propagate = functools.partial(
    pl.kernel,
    out_type=[
        jax.ShapeDtypeStruct((2 * NP, H), jnp.float32),
        jax.ShapeDtypeStruct((2 * NP, H), jnp.float32),
    ],
    mesh=plsc.VectorSubcoreMesh(core_axis_name="c", subcore_axis_name="s"),
    compiler_params=pltpu.CompilerParams(
        needs_layout_passes=False, use_tc_tiling_on_sc=False),
    scratch_types=[
        pltpu.VMEM((CN, H), jnp.float32),
        pltpu.VMEM((CN, H), jnp.float32),
        pltpu.VMEM((NT,), jnp.float32),
        pltpu.VMEM((G, BLK), jnp.int32),
        pltpu.VMEM((G, BLK), jnp.int32),
        pltpu.VMEM((S, BLK, H), jnp.float32),
        pltpu.VMEM((BLK,), jnp.float32),
        pltpu.VMEM_SHARED((NP, H), jnp.float32),
        pltpu.VMEM_SHARED((NP,), jnp.float32),
        pltpu.SemaphoreType.DMA,
        pltpu.SemaphoreType.DMA,
    ],
)(_propagate_body)


def kernel(emb, edge_index):
    row = edge_index[0].astype(jnp.int32)
    col = edge_index[1].astype(jnp.int32)
    pad = E_PAD - E
    rp = jnp.concatenate([row, jnp.zeros((pad,), jnp.int32)])
    cp = jnp.concatenate([col, jnp.full((pad,), N, jnp.int32)])
    row2d = jnp.concatenate([rp, rp + NP]).reshape(2 * RB, BLK)
    col2d = cp.reshape(RB, BLK)
    emb_flat = jnp.zeros((2 * NP, H), jnp.float32)
    emb_flat = emb_flat.at[0:N].set(emb[:, :H]).at[NP:NP + N].set(emb[:, H:])
    o_flat, _ = _propagate(emb_flat, row2d, col2d)
    return jnp.concatenate([o_flat[:N], o_flat[NP:NP + N]], axis=1)


# speedup vs baseline: 17.2393x; 1.0036x over previous
import functools

import jax
import jax.numpy as jnp
from jax import lax
from jax.experimental import pallas as pl
from jax.experimental.pallas import tpu as pltpu
from jax.experimental.pallas import tpu_sc as plsc

N = 50000
D = 64
H = 32
NUM_LAYERS = 3
ALPHA = 1.0 / (NUM_LAYERS + 1)

NC = 2
NS = 16

NP = 50176
NT = NP // NS
CN = 64
NQ = NT // CN
S = 4

E = 800000
EPT = 50176
E_PAD = EPT * NS
BLK = 128
G = 8
CB = EPT // BLK
NJ = CB // G
RB = E_PAD // BLK


def _rsqrt16(d):
    y = jnp.full((16,), 2.0 ** -11, jnp.float32)
    for k in range(9, -1, -1):
        y = jnp.where(d < 4.0 ** (k + 1), jnp.float32(2.0 ** -(k + 1)), y)
    for _ in range(6):
        y = y * (1.5 - 0.5 * d * y * y)
    return jnp.where(d > 0.5, y, 0.0)


def _propagate_body(e0, e1, row2d, col2d, o0, o1, y0, y1,
                    xb, ob, dinvv, idx_r, idx_c, rows, ones_v,
                    acc_sh, deg_sh, gsem, ssem):
    c = lax.axis_index("c")
    s = lax.axis_index("s")
    z16 = jnp.zeros((16,), jnp.float32)
    one16 = jnp.ones((16,), jnp.float32)

    def _zero_ob(i, _):
        ob[i, 0:16] = z16
        ob[i, 16:32] = z16
        return 0

    def _fill_ones(k, _):
        ones_v[pl.ds(k * 16, 16)] = one16
        return 0
    lax.fori_loop(0, BLK // 16, _fill_ones, 0)

    def _zero_dinvv(k, _):
        dinvv[pl.ds(k * 16, 16)] = z16
        return 0
    lax.fori_loop(0, NT // 16, _zero_dinvv, 0)
    pltpu.sync_copy(dinvv, deg_sh.at[pl.ds(s * NT, NT)])
    plsc.subcore_barrier()

    def _deg_chunk(j, _):
        cblk = s * CB + j * G
        pltpu.sync_copy(col2d.at[pl.ds(cblk, G)], idx_c)
        descs = [pltpu.async_copy(ones_v, deg_sh.at[idx_c.at[t]], ssem,
                                  add=True)
                 for t in range(G)]
        for dsc in descs:
            dsc.wait()
        return 0
    lax.fori_loop(0, NJ, _deg_chunk, 0)
    plsc.subcore_barrier()

    pltpu.sync_copy(deg_sh.at[pl.ds(s * NT, NT)], dinvv)

    def _dinv(k, _):
        dinvv[pl.ds(k * 16, 16)] = _rsqrt16(dinvv[pl.ds(k * 16, 16)])
        return 0
    lax.fori_loop(0, NT // 16, _dinv, 0)

    def _p0_loop(e_ref, o_ref, y_ref):
        def _p0(q, _):
            g = s * NT + q * CN
            pltpu.sync_copy(e_ref.at[pl.ds(g, CN)], xb)
            pltpu.sync_copy(xb, o_ref.at[pl.ds(g, CN)])

            def _n(i, _):
                li = q * CN + i
                dv = plsc.load_gather(dinvv,
                                      [jnp.full((16,), li, jnp.int32)])
                xb[i, 0:16] = xb[i, 0:16] * dv
                xb[i, 16:32] = xb[i, 16:32] * dv
                return 0
            lax.fori_loop(0, CN, _n, 0)
            pltpu.sync_copy(xb, y_ref.at[pl.ds(g, CN)])
            return 0
        lax.fori_loop(0, NQ, _p0, 0)

    @pl.when(c == 0)
    def _():
        _p0_loop(e0, o0, y0)

    @pl.when(c == 1)
    def _():
        _p0_loop(e1, o1, y1)

    def _edge_loop(y_ref):
        def _edge_chunk(j, _):
            cblk = s * CB + j * G
            pltpu.sync_copy(row2d.at[pl.ds(cblk, G)], idx_r)
            pltpu.sync_copy(col2d.at[pl.ds(cblk, G)], idx_c)
            gd = [None] * G
            sd = [None] * G
            for t in range(G):
                if t >= S:
                    sd[t - S].wait()
                gd[t] = pltpu.async_copy(y_ref.at[idx_r.at[t]],
                                         rows.at[t % S], gsem)
                u = t - (S - 1)
                if u >= 0:
                    gd[u].wait()
                    sd[u] = pltpu.async_copy(rows.at[u % S],
                                             acc_sh.at[idx_c.at[u]],
                                             ssem, add=True)
            for u in range(G - (S - 1), G):
                gd[u].wait()
                sd[u] = pltpu.async_copy(rows.at[u % S],
                                         acc_sh.at[idx_c.at[u]],
                                         ssem, add=True)
            for u in range(G - S, G):
                if u >= 0:
                    sd[u].wait()
            return 0
        lax.fori_loop(0, NJ, _edge_chunk, 0)

    def _node_loop(o_ref, y_ref, last):
        def _npass(q, _):
            g = s * NT + q * CN
            pltpu.sync_copy(acc_sh.at[pl.ds(g, CN)], xb)
            pltpu.sync_copy(o_ref.at[pl.ds(g, CN)], ob)

            def _n(i, _):
                li = q * CN + i
                dv = plsc.load_gather(dinvv,
                                      [jnp.full((16,), li, jnp.int32)])
                x0 = xb[i, 0:16] * dv
                x1 = xb[i, 16:32] * dv
                o0_ = ob[i, 0:16] + x0
                o1_ = ob[i, 16:32] + x1
                if last:
                    o0_ = o0_ * ALPHA
                    o1_ = o1_ * ALPHA
                else:
                    xb[i, 0:16] = x0 * dv
                    xb[i, 16:32] = x1 * dv
                ob[i, 0:16] = o0_
                ob[i, 16:32] = o1_
                return 0
            lax.fori_loop(0, CN, _n, 0)
            pltpu.sync_copy(ob, o_ref.at[pl.ds(g, CN)])
            if not last:
                pltpu.sync_copy(xb, y_ref.at[pl.ds(g, CN)])
            return 0
        lax.fori_loop(0, NQ, _npass, 0)

    for l in range(NUM_LAYERS):
        last = l == NUM_LAYERS - 1

        lax.fori_loop(0, CN, _zero_ob, 0)

        def _zero_acc(q, _):
            pltpu.sync_copy(ob, acc_sh.at[pl.ds(s * NT + q * CN, CN)])
            return 0
        lax.fori_loop(0, NQ, _zero_acc, 0)
        plsc.subcore_barrier()

        @pl.when(c == 0)
        def _():
            _edge_loop(y0)

        @pl.when(c == 1)
        def _():
            _edge_loop(y1)
        plsc.subcore_barrier()

        @pl.when(c == 0)
        def _():
            _node_loop(o0, y0, last)

        @pl.when(c == 1)
        def _():
            _node_loop(o1, y1, last)


_propagate = functools.partial(
    pl.kernel,
    out_type=[
        jax.ShapeDtypeStruct((NP, H), jnp.float32),
        jax.ShapeDtypeStruct((NP, H), jnp.float32),
        jax.ShapeDtypeStruct((NP, H), jnp.float32),
        jax.ShapeDtypeStruct((NP, H), jnp.float32),
    ],
    mesh=plsc.VectorSubcoreMesh(core_axis_name="c", subcore_axis_name="s"),
    compiler_params=pltpu.CompilerParams(
        needs_layout_passes=False, use_tc_tiling_on_sc=False),
    scratch_types=[
        pltpu.VMEM((CN, H), jnp.float32),
        pltpu.VMEM((CN, H), jnp.float32),
        pltpu.VMEM((NT,), jnp.float32),
        pltpu.VMEM((G, BLK), jnp.int32),
        pltpu.VMEM((G, BLK), jnp.int32),
        pltpu.VMEM((S, BLK, H), jnp.float32),
        pltpu.VMEM((BLK,), jnp.float32),
        pltpu.VMEM_SHARED((NP, H), jnp.float32),
        pltpu.VMEM_SHARED((NP,), jnp.float32),
        pltpu.SemaphoreType.DMA,
        pltpu.SemaphoreType.DMA,
    ],
)(_propagate_body)


def kernel(emb, edge_index):
    row = edge_index[0].astype(jnp.int32)
    col = edge_index[1].astype(jnp.int32)
    pad = E_PAD - E
    rp = jnp.concatenate([row, jnp.zeros((pad,), jnp.int32)])
    cp = jnp.concatenate([col, jnp.full((pad,), N, jnp.int32)])
    row2d = rp.reshape(RB, BLK)
    col2d = cp.reshape(RB, BLK)
    e0 = jnp.pad(emb[:, :H], ((0, NP - N), (0, 0)))
    e1 = jnp.pad(emb[:, H:], ((0, NP - N), (0, 0)))
    o0, o1, _, _ = _propagate(e0, e1, row2d, col2d)
    return jnp.concatenate([o0[:N], o1[:N]], axis=1)


# speedup vs baseline: 19.4743x; 1.1296x over previous
import functools

import jax
import jax.numpy as jnp
from jax import lax
from jax.experimental import pallas as pl
from jax.experimental.pallas import tpu as pltpu
from jax.experimental.pallas import tpu_sc as plsc

N = 50000
D = 64
H = 32
NUM_LAYERS = 3
ALPHA = 1.0 / (NUM_LAYERS + 1)

NC = 2
NS = 16

NP = 50176
NT = NP // NS
CN = 64
NQ = NT // CN
REM = N % CN
S = 4

E = 800000
EPT = 50176
E_PAD = EPT * NS
BLK = 128
G = 8
CB = EPT // BLK
NJ = CB // G
RB = E_PAD // BLK


def _rsqrt16(d):
    y = jnp.full((16,), 2.0 ** -11, jnp.float32)
    for k in range(9, -1, -1):
        y = jnp.where(d < 4.0 ** (k + 1), jnp.float32(2.0 ** -(k + 1)), y)
    for _ in range(6):
        y = y * (1.5 - 0.5 * d * y * y)
    return jnp.where(d > 0.5, y, 0.0)


def _propagate_body(emb, row2d, col2d, o, y0, y1,
                    xb, ob, dinvv, idx_r, idx_c, rows, ones_v,
                    acc_sh, deg_sh, gsem, ssem):
    c = lax.axis_index("c")
    s = lax.axis_index("s")
    z16 = jnp.zeros((16,), jnp.float32)
    one16 = jnp.ones((16,), jnp.float32)

    def _zero_ob(i, _):
        ob[i, 0:16] = z16
        ob[i, 16:32] = z16
        return 0

    def _fill_ones(k, _):
        ones_v[pl.ds(k * 16, 16)] = one16
        return 0
    lax.fori_loop(0, BLK // 16, _fill_ones, 0)

    def _zero_dinvv(k, _):
        dinvv[pl.ds(k * 16, 16)] = z16
        return 0
    lax.fori_loop(0, NT // 16, _zero_dinvv, 0)
    pltpu.sync_copy(dinvv, deg_sh.at[pl.ds(s * NT, NT)])
    plsc.subcore_barrier()

    def _deg_chunk(j, _):
        cblk = s * CB + j * G
        pltpu.sync_copy(col2d.at[pl.ds(cblk, G)], idx_c)
        descs = [pltpu.async_copy(ones_v, deg_sh.at[idx_c.at[t]], ssem,
                                  add=True)
                 for t in range(G)]
        for dsc in descs:
            dsc.wait()
        return 0
    lax.fori_loop(0, NJ, _deg_chunk, 0)
    plsc.subcore_barrier()

    pltpu.sync_copy(deg_sh.at[pl.ds(s * NT, NT)], dinvv)

    def _dinv(k, _):
        dinvv[pl.ds(k * 16, 16)] = _rsqrt16(dinvv[pl.ds(k * 16, 16)])
        return 0
    lax.fori_loop(0, NT // 16, _dinv, 0)

    def _p0_loop(col0, y_ref):
        def _p0(q, _):
            g = s * NT + q * CN

            @pl.when(g + CN <= N)
            def _():
                pltpu.sync_copy(emb.at[pl.ds(g, CN), pl.ds(col0, H)], xb)
                pltpu.sync_copy(xb, o.at[pl.ds(g, CN), pl.ds(col0, H)])

            @pl.when(jnp.logical_and(g + CN > N, g < N))
            def _():
                pltpu.sync_copy(emb.at[pl.ds(g, REM), pl.ds(col0, H)],
                                xb.at[pl.ds(0, REM)])
                pltpu.sync_copy(xb.at[pl.ds(0, REM)],
                                o.at[pl.ds(g, REM), pl.ds(col0, H)])

            def _n(i, _):
                li = q * CN + i
                dv = plsc.load_gather(dinvv,
                                      [jnp.full((16,), li, jnp.int32)])
                xb[i, 0:16] = xb[i, 0:16] * dv
                xb[i, 16:32] = xb[i, 16:32] * dv
                return 0
            lax.fori_loop(0, CN, _n, 0)
            pltpu.sync_copy(xb, y_ref.at[pl.ds(g, CN)])
            return 0
        lax.fori_loop(0, NQ, _p0, 0)

    @pl.when(c == 0)
    def _():
        _p0_loop(0, y0)

    @pl.when(c == 1)
    def _():
        _p0_loop(H, y1)

    def _edge_loop(y_ref):
        def _edge_chunk(j, _):
            cblk = s * CB + j * G
            pltpu.sync_copy(row2d.at[pl.ds(cblk, G)], idx_r)
            pltpu.sync_copy(col2d.at[pl.ds(cblk, G)], idx_c)
            gd = [None] * G
            sd = [None] * G
            for t in range(G):
                if t >= S:
                    sd[t - S].wait()
                gd[t] = pltpu.async_copy(y_ref.at[idx_r.at[t]],
                                         rows.at[t % S], gsem)
                u = t - (S - 1)
                if u >= 0:
                    gd[u].wait()
                    sd[u] = pltpu.async_copy(rows.at[u % S],
                                             acc_sh.at[idx_c.at[u]],
                                             ssem, add=True)
            for u in range(G - (S - 1), G):
                gd[u].wait()
                sd[u] = pltpu.async_copy(rows.at[u % S],
                                         acc_sh.at[idx_c.at[u]],
                                         ssem, add=True)
            for u in range(G - S, G):
                if u >= 0:
                    sd[u].wait()
            return 0
        lax.fori_loop(0, NJ, _edge_chunk, 0)

    def _node_loop(col0, y_ref, last):
        def _npass(q, _):
            g = s * NT + q * CN
            pltpu.sync_copy(acc_sh.at[pl.ds(g, CN)], xb)

            @pl.when(g + CN <= N)
            def _():
                pltpu.sync_copy(o.at[pl.ds(g, CN), pl.ds(col0, H)], ob)

            @pl.when(jnp.logical_and(g + CN > N, g < N))
            def _():
                pltpu.sync_copy(o.at[pl.ds(g, REM), pl.ds(col0, H)],
                                ob.at[pl.ds(0, REM)])

            def _n(i, _):
                li = q * CN + i
                dv = plsc.load_gather(dinvv,
                                      [jnp.full((16,), li, jnp.int32)])
                x0 = xb[i, 0:16] * dv
                x1 = xb[i, 16:32] * dv
                o0_ = ob[i, 0:16] + x0
                o1_ = ob[i, 16:32] + x1
                if last:
                    o0_ = o0_ * ALPHA
                    o1_ = o1_ * ALPHA
                else:
                    xb[i, 0:16] = x0 * dv
                    xb[i, 16:32] = x1 * dv
                ob[i, 0:16] = o0_
                ob[i, 16:32] = o1_
                return 0
            lax.fori_loop(0, CN, _n, 0)

            @pl.when(g + CN <= N)
            def _():
                pltpu.sync_copy(ob, o.at[pl.ds(g, CN), pl.ds(col0, H)])

            @pl.when(jnp.logical_and(g + CN > N, g < N))
            def _():
                pltpu.sync_copy(ob.at[pl.ds(0, REM)],
                                o.at[pl.ds(g, REM), pl.ds(col0, H)])

            if not last:
                pltpu.sync_copy(xb, y_ref.at[pl.ds(g, CN)])
            return 0
        lax.fori_loop(0, NQ, _npass, 0)

    for l in range(NUM_LAYERS):
        last = l == NUM_LAYERS - 1

        lax.fori_loop(0, CN, _zero_ob, 0)

        def _zero_acc(q, _):
            pltpu.sync_copy(ob, acc_sh.at[pl.ds(s * NT + q * CN, CN)])
            return 0
        lax.fori_loop(0, NQ, _zero_acc, 0)
        plsc.subcore_barrier()

        @pl.when(c == 0)
        def _():
            _edge_loop(y0)

        @pl.when(c == 1)
        def _():
            _edge_loop(y1)
        plsc.subcore_barrier()

        @pl.when(c == 0)
        def _():
            _node_loop(0, y0, last)

        @pl.when(c == 1)
        def _():
            _node_loop(H, y1, last)


_propagate = functools.partial(
    pl.kernel,
    out_type=[
        jax.ShapeDtypeStruct((N, D), jnp.float32),
        jax.ShapeDtypeStruct((NP, H), jnp.float32),
        jax.ShapeDtypeStruct((NP, H), jnp.float32),
    ],
    mesh=plsc.VectorSubcoreMesh(core_axis_name="c", subcore_axis_name="s"),
    compiler_params=pltpu.CompilerParams(
        needs_layout_passes=False, use_tc_tiling_on_sc=False),
    scratch_types=[
        pltpu.VMEM((CN, H), jnp.float32),
        pltpu.VMEM((CN, H), jnp.float32),
        pltpu.VMEM((NT,), jnp.float32),
        pltpu.VMEM((G, BLK), jnp.int32),
        pltpu.VMEM((G, BLK), jnp.int32),
        pltpu.VMEM((S, BLK, H), jnp.float32),
        pltpu.VMEM((BLK,), jnp.float32),
        pltpu.VMEM_SHARED((NP, H), jnp.float32),
        pltpu.VMEM_SHARED((NP,), jnp.float32),
        pltpu.SemaphoreType.DMA,
        pltpu.SemaphoreType.DMA,
    ],
)(_propagate_body)


def kernel(emb, edge_index):
    row = edge_index[0].astype(jnp.int32)
    col = edge_index[1].astype(jnp.int32)
    pad = E_PAD - E
    rp = jnp.concatenate([row, jnp.zeros((pad,), jnp.int32)])
    cp = jnp.concatenate([col, jnp.full((pad,), N, jnp.int32)])
    row2d = rp.reshape(RB, BLK)
    col2d = cp.reshape(RB, BLK)
    o, _, _ = _propagate(emb, row2d, col2d)
    return o
